# Initial kernel scaffold; baseline (speedup 1.0000x reference)
import jax
import jax.numpy as jnp
from jax.experimental import pallas as pl


def kernel(edge_index, edge_vals, embeds):
    raise NotImplementedError("write your pallas kernel here")


# speedup vs baseline: 2.6931x; 2.6931x over previous
import dataclasses
import functools

import jax
import jax.numpy as jnp
from jax import lax
from jax.experimental import pallas as pl
from jax.experimental.pallas import tpu as pltpu
from jax.experimental.pallas import tpu_sc as plsc

N_NODES = 10000
D = 128
E = 320000

NC = 2
NS = 16
LANES = 16
K = 128
CHUNKS = 80
EPT = K * CHUNKS
E_PAD = EPT * NC * NS
N_ACC = 10240
ZR = 8
OUT_PT = N_ACC // NS


def _sc_compiler_params():
    cp = pltpu.CompilerParams()
    if "needs_layout_passes" in pltpu.CompilerParams.__dataclass_fields__:
        cp = dataclasses.replace(cp, needs_layout_passes=False)
    return cp


def _sc_partial(rows2d, cols, vals, embeds):
    mesh = plsc.VectorSubcoreMesh(core_axis_name="core", subcore_axis_name="subcore")

    @functools.partial(
        pl.kernel,
        compiler_params=_sc_compiler_params(),
        out_type=jax.ShapeDtypeStruct((NC, N_ACC, D), jnp.float32),
        mesh=mesh,
        scratch_types=[
            pltpu.VMEM((1, K), jnp.int32),
            pltpu.VMEM((K,), jnp.int32),
            pltpu.VMEM((K,), jnp.float32),
            pltpu.VMEM((K, D), jnp.float32),
            pltpu.VMEM((ZR, D), jnp.float32),
            pltpu.VMEM_SHARED((N_ACC, D), jnp.float32),
            pltpu.SemaphoreType.DMA,
        ],
    )
    def body(rows_hbm, cols_hbm, vals_hbm, emb_hbm, out_hbm,
             rows_c, cols_c, vals_c, msg_v, zero_v, acc, sem):
        cid = lax.axis_index("core")
        sid = lax.axis_index("subcore")
        wid = cid * NS + sid
        base = wid * EPT

        @pl.loop(0, ZR)
        def _(r):
            for f in range(D // LANES):
                zero_v[r, pl.ds(f * LANES, LANES)] = jnp.zeros((LANES,), jnp.float32)

        @pl.loop(0, N_ACC // NS // ZR)
        def _(kk):
            pltpu.sync_copy(zero_v, acc.at[pl.ds(sid * (N_ACC // NS) + kk * ZR, ZR)])

        plsc.subcore_barrier()

        @pl.loop(0, CHUNKS)
        def _(j):
            pltpu.sync_copy(cols_hbm.at[pl.ds(base + j * K, K)], cols_c)
            pltpu.sync_copy(vals_hbm.at[pl.ds(base + j * K, K)], vals_c)
            pltpu.sync_copy(rows_hbm.at[pl.ds(wid * CHUNKS + j, 1)], rows_c)
            pltpu.async_copy(emb_hbm.at[cols_c], msg_v, sem).wait()

            @pl.loop(0, K)
            def _(e):
                vb = plsc.load_gather(vals_c, [jnp.full((LANES,), e, jnp.int32)])
                for f in range(D // LANES):
                    sl = pl.ds(f * LANES, LANES)
                    msg_v[e, sl] = msg_v[e, sl] * vb

            pltpu.sync_copy(msg_v, acc.at[rows_c.at[0]], add=True)

        plsc.subcore_barrier()

        pltpu.sync_copy(
            acc.at[pl.ds(sid * OUT_PT, OUT_PT)],
            out_hbm.at[cid, pl.ds(sid * OUT_PT, OUT_PT)],
        )

    return body(rows2d, cols, vals, embeds)


def _merge(partial):
    def add_body(a_ref, b_ref, o_ref):
        o_ref[...] = a_ref[0] + b_ref[0]

    return pl.pallas_call(
        add_body,
        grid=(10,),
        in_specs=[
            pl.BlockSpec((1, N_NODES // 10, D), lambda i: (0, i, 0)),
            pl.BlockSpec((1, N_NODES // 10, D), lambda i: (1, i, 0)),
        ],
        out_specs=pl.BlockSpec((N_NODES // 10, D), lambda i: (i, 0)),
        out_shape=jax.ShapeDtypeStruct((N_NODES, D), jnp.float32),
    )(partial, partial)


def kernel(edge_index, edge_vals, embeds):
    pad = E_PAD - E
    rows = jnp.pad(edge_index[0], (0, pad)).reshape(NC * NS * CHUNKS, K)
    cols = jnp.pad(edge_index[1], (0, pad))
    vals = jnp.pad(edge_vals, (0, pad))
    partial = _sc_partial(rows, cols, vals, embeds)
    return _merge(partial)


# speedup vs baseline: 3.9426x; 1.4640x over previous
import dataclasses
import functools

import jax
import jax.numpy as jnp
from jax import lax
from jax.experimental import pallas as pl
from jax.experimental.pallas import tpu as pltpu
from jax.experimental.pallas import tpu_sc as plsc

N_NODES = 10000
D = 128
E = 320000

NC = 2
NS = 16
LANES = 16
K = 128
CHUNKS = 80
EPT = K * CHUNKS
E_PAD = EPT * NC * NS
N_ACC = 10240
ZR = 8
OUT_PT = N_ACC // NS


def _sc_compiler_params():
    cp = pltpu.CompilerParams()
    if "needs_layout_passes" in pltpu.CompilerParams.__dataclass_fields__:
        cp = dataclasses.replace(cp, needs_layout_passes=False)
    return cp


def _sc_partial(edata, embeds):
    mesh = plsc.VectorSubcoreMesh(core_axis_name="core", subcore_axis_name="subcore")

    @functools.partial(
        pl.kernel,
        compiler_params=_sc_compiler_params(),
        out_type=jax.ShapeDtypeStruct((NC, N_ACC, D), jnp.float32),
        mesh=mesh,
        scratch_types=[
            pltpu.VMEM((1, 3, K), jnp.int32),
            pltpu.VMEM((1, 3, K), jnp.int32),
            pltpu.VMEM((K, D), jnp.float32),
            pltpu.VMEM((K, D), jnp.float32),
            pltpu.VMEM((ZR, D), jnp.float32),
            pltpu.VMEM_SHARED((N_ACC, D), jnp.float32),
            pltpu.SemaphoreType.DMA,
            pltpu.SemaphoreType.DMA,
            pltpu.SemaphoreType.DMA,
            pltpu.SemaphoreType.DMA,
        ],
    )
    def body(edata_hbm, emb_hbm, out_hbm,
             ibuf0, ibuf1, msg0, msg1, zero_v, acc, si0, si1, sg0, sg1):
        cid = lax.axis_index("core")
        sid = lax.axis_index("subcore")
        wid = cid * NS + sid
        cbase = wid * CHUNKS

        def idx_cp(j, ibuf, sem):
            return pltpu.make_async_copy(
                edata_hbm.at[pl.ds(cbase + j, 1)], ibuf, sem)

        def gather_cp(ibuf, msg, sem):
            return pltpu.make_async_copy(emb_hbm.at[ibuf.at[0, 1]], msg, sem)

        @pl.loop(0, ZR)
        def _(r):
            for f in range(D // LANES):
                zero_v[r, pl.ds(f * LANES, LANES)] = jnp.zeros((LANES,), jnp.float32)

        @pl.loop(0, N_ACC // NS // ZR)
        def _(kk):
            pltpu.sync_copy(zero_v, acc.at[pl.ds(sid * (N_ACC // NS) + kk * ZR, ZR)])

        plsc.subcore_barrier()

        bufs = ((ibuf0, msg0, si0, sg0), (ibuf1, msg1, si1, sg1))

        idx_cp(0, ibuf0, si0).start()
        idx_cp(1, ibuf1, si1).start()
        idx_cp(0, ibuf0, si0).wait()
        gather_cp(ibuf0, msg0, sg0).start()

        @pl.loop(0, CHUNKS // 2)
        def _(jj):
            for b in range(2):
                j = jj * 2 + b
                ib, mb, si, sg = bufs[b]
                io, mo, sio, sgo = bufs[1 - b]

                @pl.when(j + 1 < CHUNKS)
                def _():
                    idx_cp(j + 1, io, sio).wait()
                    gather_cp(io, mo, sgo).start()

                gather_cp(ib, mb, sg).wait()

                @pl.loop(0, K)
                def _(e):
                    vb = plsc.bitcast(
                        plsc.load_gather(
                            ib,
                            [jnp.zeros((LANES,), jnp.int32),
                             jnp.full((LANES,), 2, jnp.int32),
                             jnp.full((LANES,), e, jnp.int32)]),
                        jnp.float32)
                    for f in range(D // LANES):
                        sl = pl.ds(f * LANES, LANES)
                        mb[e, sl] = mb[e, sl] * vb

                pltpu.sync_copy(mb, acc.at[ib.at[0, 0]], add=True)

                @pl.when(j + 2 < CHUNKS)
                def _():
                    idx_cp(j + 2, ib, si).start()

        plsc.subcore_barrier()

        pltpu.sync_copy(
            acc.at[pl.ds(sid * OUT_PT, OUT_PT)],
            out_hbm.at[cid, pl.ds(sid * OUT_PT, OUT_PT)],
        )

    return body(edata, embeds)


def _merge(partial):
    def add_body(a_ref, b_ref, o_ref):
        o_ref[...] = a_ref[0] + b_ref[0]

    return pl.pallas_call(
        add_body,
        grid=(10,),
        in_specs=[
            pl.BlockSpec((1, N_NODES // 10, D), lambda i: (0, i, 0)),
            pl.BlockSpec((1, N_NODES // 10, D), lambda i: (1, i, 0)),
        ],
        out_specs=pl.BlockSpec((N_NODES // 10, D), lambda i: (i, 0)),
        out_shape=jax.ShapeDtypeStruct((N_NODES, D), jnp.float32),
    )(partial, partial)


def kernel(edge_index, edge_vals, embeds):
    pad = E_PAD - E
    rows = jnp.pad(edge_index[0], (0, pad)).reshape(-1, 1, K)
    cols = jnp.pad(edge_index[1], (0, pad)).reshape(-1, 1, K)
    vals = lax.bitcast_convert_type(
        jnp.pad(edge_vals, (0, pad)), jnp.int32).reshape(-1, 1, K)
    edata = jnp.concatenate([rows, cols, vals], axis=1)
    partial = _sc_partial(edata, embeds)
    return _merge(partial)


# speedup vs baseline: 3.9455x; 1.0007x over previous
import dataclasses
import functools

import jax
import jax.numpy as jnp
from jax import lax
from jax.experimental import pallas as pl
from jax.experimental.pallas import tpu as pltpu
from jax.experimental.pallas import tpu_sc as plsc

N_NODES = 10000
D = 128
E = 320000

NC = 2
NS = 16
LANES = 16
K = 128
CHUNKS = 80
EPT = K * CHUNKS
E_PAD = EPT * NC * NS
N_ACC = 10240
ZR = 8
OUT_PT = N_ACC // NS


def _sc_compiler_params():
    cp = pltpu.CompilerParams()
    if "needs_layout_passes" in pltpu.CompilerParams.__dataclass_fields__:
        cp = dataclasses.replace(cp, needs_layout_passes=False)
    return cp


def _sc_partial(edata, embeds):
    mesh = plsc.VectorSubcoreMesh(core_axis_name="core", subcore_axis_name="subcore")

    @functools.partial(
        pl.kernel,
        compiler_params=_sc_compiler_params(),
        out_type=jax.ShapeDtypeStruct((NC, N_ACC, D), jnp.float32),
        mesh=mesh,
        scratch_types=[
            pltpu.VMEM((1, 3, K), jnp.int32),
            pltpu.VMEM((1, 3, K), jnp.int32),
            pltpu.VMEM((1, K), jnp.int32),
            pltpu.VMEM((1, K), jnp.int32),
            pltpu.VMEM((K, D), jnp.float32),
            pltpu.VMEM((K, D), jnp.float32),
            pltpu.VMEM((ZR, D), jnp.float32),
            pltpu.VMEM_SHARED((N_ACC, D), jnp.float32),
            pltpu.SemaphoreType.DMA,
            pltpu.SemaphoreType.DMA,
            pltpu.SemaphoreType.DMA,
            pltpu.SemaphoreType.DMA,
            pltpu.SemaphoreType.DMA,
            pltpu.SemaphoreType.DMA,
        ],
    )
    def body(edata_hbm, emb_hbm, out_hbm,
             ibuf0, ibuf1, rb0, rb1, msg0, msg1, zero_v, acc,
             si0, si1, sg0, sg1, ss0, ss1):
        cid = lax.axis_index("core")
        sid = lax.axis_index("subcore")
        wid = cid * NS + sid
        cbase = wid * CHUNKS

        def idx_cp(j, ibuf, sem):
            return pltpu.make_async_copy(
                edata_hbm.at[pl.ds(cbase + j, 1)], ibuf, sem)

        def gather_cp(ibuf, msg, sem):
            return pltpu.make_async_copy(emb_hbm.at[ibuf.at[0, 1]], msg, sem)

        def scatter_start(msg, rb, sem):
            pltpu.async_copy(msg, acc.at[rb.at[0]], sem, add=True)

        def scatter_wait(msg, rb, sem):
            pltpu.make_async_copy(msg, acc.at[rb.at[0]], sem).wait()

        @pl.loop(0, ZR)
        def _(r):
            for f in range(D // LANES):
                zero_v[r, pl.ds(f * LANES, LANES)] = jnp.zeros((LANES,), jnp.float32)

        @pl.loop(0, N_ACC // NS // ZR)
        def _(kk):
            pltpu.sync_copy(zero_v, acc.at[pl.ds(sid * (N_ACC // NS) + kk * ZR, ZR)])

        plsc.subcore_barrier()

        bufs = ((ibuf0, rb0, msg0, si0, sg0, ss0), (ibuf1, rb1, msg1, si1, sg1, ss1))

        idx_cp(0, ibuf0, si0).start()
        idx_cp(1, ibuf1, si1).start()
        idx_cp(0, ibuf0, si0).wait()
        gather_cp(ibuf0, msg0, sg0).start()

        UN = 8

        @pl.loop(0, CHUNKS // 2)
        def _(jj):
            for b in range(2):
                j = jj * 2 + b
                ib, rb, mb, si, sg, ss = bufs[b]
                io, ro, mo, sio, sgo, sso = bufs[1 - b]

                @pl.when(j + 1 < CHUNKS)
                def _():
                    idx_cp(j + 1, io, sio).wait()

                    @pl.when(j >= 1)
                    def _():
                        scatter_wait(mo, ro, sso)

                    gather_cp(io, mo, sgo).start()

                gather_cp(ib, mb, sg).wait()

                @pl.loop(0, K // UN)
                def _(eb):
                    for u in range(UN):
                        e = eb * UN + u
                        vb = plsc.bitcast(
                            plsc.load_gather(
                                ib,
                                [jnp.zeros((LANES,), jnp.int32),
                                 jnp.full((LANES,), 2, jnp.int32),
                                 jnp.full((LANES,), e, jnp.int32)]),
                            jnp.float32)
                        for f in range(D // LANES):
                            sl = pl.ds(f * LANES, LANES)
                            mb[e, sl] = mb[e, sl] * vb

                for f in range(K // LANES):
                    sl = pl.ds(f * LANES, LANES)
                    rb[0, sl] = ib[0, 0, sl]

                scatter_start(mb, rb, ss)

                @pl.when(j + 2 < CHUNKS)
                def _():
                    idx_cp(j + 2, ib, si).start()

        scatter_wait(msg0, rb0, ss0)
        scatter_wait(msg1, rb1, ss1)

        plsc.subcore_barrier()

        pltpu.sync_copy(
            acc.at[pl.ds(sid * OUT_PT, OUT_PT)],
            out_hbm.at[cid, pl.ds(sid * OUT_PT, OUT_PT)],
        )

    return body(edata, embeds)


def _merge(partial):
    def add_body(a_ref, b_ref, o_ref):
        o_ref[...] = a_ref[0] + b_ref[0]

    return pl.pallas_call(
        add_body,
        grid=(10,),
        in_specs=[
            pl.BlockSpec((1, N_NODES // 10, D), lambda i: (0, i, 0)),
            pl.BlockSpec((1, N_NODES // 10, D), lambda i: (1, i, 0)),
        ],
        out_specs=pl.BlockSpec((N_NODES // 10, D), lambda i: (i, 0)),
        out_shape=jax.ShapeDtypeStruct((N_NODES, D), jnp.float32),
    )(partial, partial)


def kernel(edge_index, edge_vals, embeds):
    pad = E_PAD - E
    rows = jnp.pad(edge_index[0], (0, pad)).reshape(-1, 1, K)
    cols = jnp.pad(edge_index[1], (0, pad)).reshape(-1, 1, K)
    vals = lax.bitcast_convert_type(
        jnp.pad(edge_vals, (0, pad)), jnp.int32).reshape(-1, 1, K)
    edata = jnp.concatenate([rows, cols, vals], axis=1)
    partial = _sc_partial(edata, embeds)
    return _merge(partial)


# speedup vs baseline: 9.6875x; 2.4553x over previous
import dataclasses
import functools

import jax
import jax.numpy as jnp
from jax import lax
from jax.experimental import pallas as pl
from jax.experimental.pallas import tpu as pltpu
from jax.experimental.pallas import tpu_sc as plsc

N_NODES = 10000
D = 128
E = 320000

NC = 2
NS = 16
LANES = 16
K = 128
CHUNKS = 80
EPT = K * CHUNKS
E_PAD = EPT * NC * NS
N_ACC = 10240
ZR = 8
OUT_PT = N_ACC // NS


def _sc_compiler_params():
    cp = pltpu.CompilerParams()
    if "needs_layout_passes" in pltpu.CompilerParams.__dataclass_fields__:
        cp = dataclasses.replace(cp, needs_layout_passes=False)
    return cp


def _sc_partial(edata, embeds):
    mesh = plsc.VectorSubcoreMesh(core_axis_name="core", subcore_axis_name="subcore")

    @functools.partial(
        pl.kernel,
        compiler_params=_sc_compiler_params(),
        out_type=jax.ShapeDtypeStruct((NC, N_ACC, D), jnp.float32),
        mesh=mesh,
        scratch_types=[
            pltpu.VMEM((1, 3, K), jnp.int32),
            pltpu.VMEM((1, 3, K), jnp.int32),
            pltpu.VMEM((1, K), jnp.int32),
            pltpu.VMEM((1, K), jnp.int32),
            pltpu.VMEM((K, D), jnp.float32),
            pltpu.VMEM((K, D), jnp.float32),
            pltpu.VMEM((ZR, D), jnp.float32),
            pltpu.VMEM_SHARED((N_ACC, D), jnp.float32),
            pltpu.SemaphoreType.DMA,
            pltpu.SemaphoreType.DMA,
            pltpu.SemaphoreType.DMA,
            pltpu.SemaphoreType.DMA,
            pltpu.SemaphoreType.DMA,
            pltpu.SemaphoreType.DMA,
        ],
    )
    def body(edata_hbm, emb_hbm, out_hbm,
             ibuf0, ibuf1, rb0, rb1, msg0, msg1, zero_v, acc,
             si0, si1, sg0, sg1, ss0, ss1):
        cid = lax.axis_index("core")
        sid = lax.axis_index("subcore")
        wid = cid * NS + sid
        cbase = wid * CHUNKS

        def idx_cp(j, ibuf, sem):
            return pltpu.make_async_copy(
                edata_hbm.at[pl.ds(cbase + j, 1)], ibuf, sem)

        def gather_cp(ibuf, msg, sem):
            return pltpu.make_async_copy(emb_hbm.at[ibuf.at[0, 1]], msg, sem)

        def scatter_start(msg, rb, sem):
            pltpu.async_copy(msg, acc.at[rb.at[0]], sem, add=True)

        def scatter_wait(msg, rb, sem):
            pltpu.make_async_copy(msg, acc.at[rb.at[0]], sem).wait()

        @pl.loop(0, ZR)
        def _(r):
            for f in range(D // LANES):
                zero_v[r, pl.ds(f * LANES, LANES)] = jnp.zeros((LANES,), jnp.float32)

        @pl.loop(0, N_ACC // NS // ZR)
        def _(kk):
            pltpu.sync_copy(zero_v, acc.at[pl.ds(sid * (N_ACC // NS) + kk * ZR, ZR)])

        plsc.subcore_barrier()

        bufs = ((ibuf0, rb0, msg0, si0, sg0, ss0), (ibuf1, rb1, msg1, si1, sg1, ss1))

        idx_cp(0, ibuf0, si0).start()
        idx_cp(1, ibuf1, si1).start()
        idx_cp(0, ibuf0, si0).wait()
        gather_cp(ibuf0, msg0, sg0).start()

        UN = 8

        @pl.loop(0, CHUNKS // 2)
        def _(jj):
            for b in range(2):
                j = jj * 2 + b
                ib, rb, mb, si, sg, ss = bufs[b]
                io, ro, mo, sio, sgo, sso = bufs[1 - b]

                @pl.when(j + 1 < CHUNKS)
                def _():
                    idx_cp(j + 1, io, sio).wait()

                    @pl.when(j >= 1)
                    def _():
                        scatter_wait(mo, ro, sso)

                    gather_cp(io, mo, sgo).start()

                gather_cp(ib, mb, sg).wait()

                @pl.loop(0, K // UN)
                def _(eb):
                    for u in range(UN):
                        e = eb * UN + u
                        vb = plsc.bitcast(
                            plsc.load_gather(
                                ib,
                                [jnp.zeros((LANES,), jnp.int32),
                                 jnp.full((LANES,), 2, jnp.int32),
                                 jnp.full((LANES,), e, jnp.int32)]),
                            jnp.float32)
                        for f in range(D // LANES):
                            sl = pl.ds(f * LANES, LANES)
                            mb[e, sl] = mb[e, sl] * vb

                for f in range(K // LANES):
                    sl = pl.ds(f * LANES, LANES)
                    rb[0, sl] = ib[0, 0, sl]

                scatter_start(mb, rb, ss)

                @pl.when(j + 2 < CHUNKS)
                def _():
                    idx_cp(j + 2, ib, si).start()

        scatter_wait(msg0, rb0, ss0)
        scatter_wait(msg1, rb1, ss1)

        plsc.subcore_barrier()

        pltpu.sync_copy(
            acc.at[pl.ds(sid * OUT_PT, OUT_PT)],
            out_hbm.at[cid, pl.ds(sid * OUT_PT, OUT_PT)],
        )

    return body(edata, embeds)


def _merge(partial):
    def add_body(a_ref, b_ref, o_ref):
        o_ref[...] = a_ref[0] + b_ref[0]

    return pl.pallas_call(
        add_body,
        grid=(10,),
        in_specs=[
            pl.BlockSpec((1, N_NODES // 10, D), lambda i: (0, i, 0)),
            pl.BlockSpec((1, N_NODES // 10, D), lambda i: (1, i, 0)),
        ],
        out_specs=pl.BlockSpec((N_NODES // 10, D), lambda i: (i, 0)),
        out_shape=jax.ShapeDtypeStruct((N_NODES, D), jnp.float32),
    )(partial, partial)


def kernel(edge_index, edge_vals, embeds):
    pad = E_PAD - E
    fill = jnp.arange(pad, dtype=jnp.int32) % N_NODES
    rows = jnp.concatenate([edge_index[0], fill]).reshape(-1, 1, K)
    cols = jnp.concatenate([edge_index[1], fill]).reshape(-1, 1, K)
    vals = lax.bitcast_convert_type(
        jnp.pad(edge_vals, (0, pad)), jnp.int32).reshape(-1, 1, K)
    edata = jnp.concatenate([rows, cols, vals], axis=1)
    partial = _sc_partial(edata, embeds)
    return _merge(partial)


# speedup vs baseline: 11.4345x; 1.1803x over previous
import dataclasses
import functools

import jax
import jax.numpy as jnp
from jax import lax
from jax.experimental import pallas as pl
from jax.experimental.pallas import tpu as pltpu
from jax.experimental.pallas import tpu_sc as plsc

N_NODES = 10000
D = 128
E = 320000

NC = 2
NS = 16
LANES = 16
K = 128
CHUNKS = 80
EPT = K * CHUNKS
E_PAD = EPT * NC * NS
N_ACC = 10240
ZR = 8
OUT_PT = N_ACC // NS


def _sc_compiler_params():
    cp = pltpu.CompilerParams()
    if "needs_layout_passes" in pltpu.CompilerParams.__dataclass_fields__:
        cp = dataclasses.replace(cp, needs_layout_passes=False)
    return cp


def _sc_partial(edata, embeds):
    mesh = plsc.VectorSubcoreMesh(core_axis_name="core", subcore_axis_name="subcore")

    @functools.partial(
        pl.kernel,
        compiler_params=_sc_compiler_params(),
        out_type=jax.ShapeDtypeStruct((NC, N_ACC, D), jnp.float32),
        mesh=mesh,
        scratch_types=[
            pltpu.VMEM((1, 3, K), jnp.int32),
            pltpu.VMEM((1, 3, K), jnp.int32),
            pltpu.VMEM((1, K), jnp.int32),
            pltpu.VMEM((1, K), jnp.int32),
            pltpu.VMEM((K, D), jnp.float32),
            pltpu.VMEM((K, D), jnp.float32),
            pltpu.VMEM_SHARED((N_ACC, D), jnp.float32),
            pltpu.SemaphoreType.DMA,
            pltpu.SemaphoreType.DMA,
            pltpu.SemaphoreType.DMA,
            pltpu.SemaphoreType.DMA,
            pltpu.SemaphoreType.DMA,
            pltpu.SemaphoreType.DMA,
            pltpu.SemaphoreType.DMA,
        ],
    )
    def body(edata_hbm, zeros_hbm, emb_hbm, out_hbm,
             ibuf0, ibuf1, rb0, rb1, msg0, msg1, acc,
             si0, si1, sg0, sg1, ss0, ss1, sz):
        cid = lax.axis_index("core")
        sid = lax.axis_index("subcore")
        wid = cid * NS + sid
        cbase = wid * CHUNKS

        def idx_cp(j, ibuf, sem):
            return pltpu.make_async_copy(
                edata_hbm.at[pl.ds(cbase + j, 1)], ibuf, sem)

        def gather_cp(ibuf, msg, sem):
            return pltpu.make_async_copy(emb_hbm.at[ibuf.at[0, 1]], msg, sem)

        def scatter_start(msg, rb, sem):
            pltpu.async_copy(msg, acc.at[rb.at[0]], sem, add=True)

        def scatter_wait(msg, rb, sem):
            pltpu.make_async_copy(msg, acc.at[rb.at[0]], sem).wait()

        zslice = pl.ds(sid * OUT_PT, OUT_PT)
        zcp = pltpu.make_async_copy(zeros_hbm.at[zslice], acc.at[zslice], sz)
        zcp.start()

        bufs = ((ibuf0, rb0, msg0, si0, sg0, ss0), (ibuf1, rb1, msg1, si1, sg1, ss1))

        idx_cp(0, ibuf0, si0).start()
        idx_cp(1, ibuf1, si1).start()
        idx_cp(0, ibuf0, si0).wait()
        gather_cp(ibuf0, msg0, sg0).start()

        zcp.wait()
        plsc.subcore_barrier()


        @pl.loop(0, CHUNKS // 2)
        def _(jj):
            for b in range(2):
                j = jj * 2 + b
                ib, rb, mb, si, sg, ss = bufs[b]
                io, ro, mo, sio, sgo, sso = bufs[1 - b]

                @pl.when(j + 1 < CHUNKS)
                def _():
                    idx_cp(j + 1, io, sio).wait()

                    @pl.when(j >= 1)
                    def _():
                        scatter_wait(mo, ro, sso)

                    gather_cp(io, mo, sgo).start()

                gather_cp(ib, mb, sg).wait()

                @plsc.parallel_loop(0, K // LANES, unroll=2)
                def _(g):
                    vv = plsc.bitcast(ib[0, 2, pl.ds(g * LANES, LANES)],
                                      jnp.float32)
                    for u in range(LANES):
                        vb = vv.at[jnp.full((LANES,), u, jnp.int32)].get(
                            mode="promise_in_bounds")
                        e = g * LANES + u
                        for f in range(D // LANES):
                            sl = pl.ds(f * LANES, LANES)
                            mb[e, sl] = mb[e, sl] * vb

                for f in range(K // LANES):
                    sl = pl.ds(f * LANES, LANES)
                    rb[0, sl] = ib[0, 0, sl]

                scatter_start(mb, rb, ss)

                @pl.when(j + 2 < CHUNKS)
                def _():
                    idx_cp(j + 2, ib, si).start()

        scatter_wait(msg0, rb0, ss0)
        scatter_wait(msg1, rb1, ss1)

        plsc.subcore_barrier()

        pltpu.sync_copy(
            acc.at[pl.ds(sid * OUT_PT, OUT_PT)],
            out_hbm.at[cid, pl.ds(sid * OUT_PT, OUT_PT)],
        )

    return body(edata, jnp.zeros((N_ACC, D), jnp.float32), embeds)


def _merge(partial):
    def add_body(a_ref, b_ref, o_ref):
        o_ref[...] = a_ref[0] + b_ref[0]

    return pl.pallas_call(
        add_body,
        grid=(10,),
        in_specs=[
            pl.BlockSpec((1, N_NODES // 10, D), lambda i: (0, i, 0)),
            pl.BlockSpec((1, N_NODES // 10, D), lambda i: (1, i, 0)),
        ],
        out_specs=pl.BlockSpec((N_NODES // 10, D), lambda i: (i, 0)),
        out_shape=jax.ShapeDtypeStruct((N_NODES, D), jnp.float32),
    )(partial, partial)


def kernel(edge_index, edge_vals, embeds):
    pad = E_PAD - E
    fill = jnp.arange(pad, dtype=jnp.int32) % N_NODES
    rows = jnp.concatenate([edge_index[0], fill]).reshape(-1, 1, K)
    cols = jnp.concatenate([edge_index[1], fill]).reshape(-1, 1, K)
    vals = lax.bitcast_convert_type(
        jnp.pad(edge_vals, (0, pad)), jnp.int32).reshape(-1, 1, K)
    edata = jnp.concatenate([rows, cols, vals], axis=1)
    partial = _sc_partial(edata, embeds)
    return _merge(partial)
